# baseline (speedup 1.0000x reference)
import functools

import jax
import jax.numpy as jnp
from jax import lax
from jax.experimental import pallas as pl
from jax.experimental.pallas import tpu as pltpu
from jax.experimental.pallas import tpu_sc as plsc

N = 10000
D = 128
B = 16
NC = 2
NS = 16
NW = NC * NS
CH = 128
NPAD = 10112
TRASH = N

_HI = lax.Precision.DEFAULT


_RB = 2000


def _lin2_body(x_ref, w0_ref, b0_ref, w1_ref, b1_ref, v0_ref, v1_ref):
    xb = x_ref[...]
    v0_ref[...] = jnp.dot(xb, w0_ref[...], preferred_element_type=jnp.float32,
                          precision=_HI) + b0_ref[...]
    v1_ref[...] = jnp.dot(xb, w1_ref[...], preferred_element_type=jnp.float32,
                          precision=_HI) + b1_ref[...]


_lin2 = pl.pallas_call(
    _lin2_body,
    grid=(N // _RB,),
    in_specs=[
        pl.BlockSpec((_RB, D), lambda i: (i, 0)),
        pl.BlockSpec((D, D), lambda i: (0, 0)),
        pl.BlockSpec((1, D), lambda i: (0, 0)),
        pl.BlockSpec((D, D), lambda i: (0, 0)),
        pl.BlockSpec((1, D), lambda i: (0, 0)),
    ],
    out_specs=[
        pl.BlockSpec((_RB, D), lambda i: (i, 0)),
        pl.BlockSpec((_RB, D), lambda i: (i, 0)),
    ],
    out_shape=[
        jax.ShapeDtypeStruct((N, D), jnp.float32),
        jax.ShapeDtypeStruct((N, D), jnp.float32),
    ],
)


def _layer2_body(v0_ref, agg_ref, w0_ref, b0_ref, w1_ref, b1_ref,
                 o0_ref, o1_ref):
    v = jnp.maximum(v0_ref[...] + agg_ref[0] + agg_ref[1], 0.0)
    o0_ref[...] = jnp.dot(v, w0_ref[...], preferred_element_type=jnp.float32,
                          precision=_HI) + b0_ref[...]
    o1_ref[...] = jnp.dot(v, w1_ref[...], preferred_element_type=jnp.float32,
                          precision=_HI) + b1_ref[...]


_layer2 = pl.pallas_call(
    _layer2_body,
    grid=(N // _RB,),
    in_specs=[
        pl.BlockSpec((_RB, D), lambda i: (i, 0)),
        pl.BlockSpec((NC, _RB, D), lambda i: (0, i, 0)),
        pl.BlockSpec((D, D), lambda i: (0, 0)),
        pl.BlockSpec((1, D), lambda i: (0, 0)),
        pl.BlockSpec((D, D), lambda i: (0, 0)),
        pl.BlockSpec((1, D), lambda i: (0, 0)),
    ],
    out_specs=[
        pl.BlockSpec((_RB, D), lambda i: (i, 0)),
        pl.BlockSpec((_RB, D), lambda i: (i, 0)),
    ],
    out_shape=[
        jax.ShapeDtypeStruct((N, D), jnp.float32),
        jax.ShapeDtypeStruct((N, D), jnp.float32),
    ],
)


def _pool_head_body(v0_ref, agg_ref, vi_ref, fc1w_ref, fc1b_ref,
                    wcat_ref, bcat_ref, out_ref):
    v2 = jnp.maximum(v0_ref[...] + agg_ref[0, :N] + agg_ref[1, :N], 0.0)
    vi = vi_ref[...]
    seg = lax.broadcasted_iota(jnp.int32, (B, N), 0)
    onehot = (seg == vi).astype(jnp.float32)
    counts = jnp.sum(onehot, axis=1)
    maxc = jnp.max(counts)
    pooled = jnp.dot(onehot, v2, preferred_element_type=jnp.float32,
                     precision=lax.Precision.HIGHEST) / maxc
    h = jnp.maximum(jnp.dot(pooled, fc1w_ref[...],
                            preferred_element_type=jnp.float32,
                            precision=_HI) + fc1b_ref[...], 0.0)
    out_ref[...] = jnp.dot(h, wcat_ref[...],
                           preferred_element_type=jnp.float32,
                           precision=_HI) + bcat_ref[...]


_pool_head = pl.pallas_call(
    _pool_head_body,
    out_shape=jax.ShapeDtypeStruct((B, D), jnp.float32),
)


def _edge_agg_body(cpt, v1_hbm, msg_hbm, zeros_hbm, out_hbm,
                   midx0, midx1, rows0, rows1, acc,
                   isem0, isem1, gsem0, gsem1):
    c = lax.axis_index("c")
    s = lax.axis_index("s")
    wid = c * NS + s
    zr = NPAD // NS
    pltpu.sync_copy(zeros_hbm.at[pl.ds(s * zr, zr)], acc.at[pl.ds(s * zr, zr)])
    plsc.subcore_barrier()

    midx = (midx0, midx1)
    rows = (rows0, rows1)
    isem = (isem0, isem1)
    gsem = (gsem0, gsem1)
    base = wid * cpt

    pltpu.sync_copy(msg_hbm.at[base], midx0)
    pltpu.async_copy(msg_hbm.at[base + 1], midx1, isem1)
    pltpu.async_copy(v1_hbm.at[midx0.at[0]], rows0, gsem0)

    def pair(j, carry):
        for k in range(2):
            i = 2 * j + k
            b, nb = k, 1 - k
            pltpu.make_async_copy(msg_hbm.at[base], midx[nb], isem[nb]).wait()
            pltpu.async_copy(v1_hbm.at[midx[nb].at[0]], rows[nb], gsem[nb])
            pltpu.make_async_copy(v1_hbm.at[pl.ds(0, CH)], rows[b],
                                  gsem[b]).wait()
            pltpu.sync_copy(rows[b], acc.at[midx[b].at[1]], add=True)
            nxt = jnp.minimum(i + 2, cpt - 1)
            pltpu.async_copy(msg_hbm.at[base + nxt], midx[b], isem[b])
        return carry

    lax.fori_loop(0, cpt // 2, pair, 0)
    pltpu.make_async_copy(v1_hbm.at[pl.ds(0, CH)], rows0, gsem0).wait()
    pltpu.make_async_copy(msg_hbm.at[base], midx1, isem1).wait()
    plsc.subcore_barrier()
    pltpu.sync_copy(acc.at[pl.ds(s * zr, zr)],
                    out_hbm.at[c].at[pl.ds(s * zr, zr)])


@functools.lru_cache(maxsize=None)
def _make_edge_agg(cpt):
    return pl.kernel(
        functools.partial(_edge_agg_body, cpt),
        out_type=jax.ShapeDtypeStruct((NC, NPAD, D), jnp.float32),
        mesh=plsc.VectorSubcoreMesh(core_axis_name="c", subcore_axis_name="s"),
        scratch_types=[
            pltpu.VMEM((2, CH), jnp.int32),
            pltpu.VMEM((2, CH), jnp.int32),
            pltpu.VMEM((CH, D), jnp.float32),
            pltpu.VMEM((CH, D), jnp.float32),
            pltpu.VMEM_SHARED((NPAD, D), jnp.float32),
            pltpu.SemaphoreType.DMA,
            pltpu.SemaphoreType.DMA,
            pltpu.SemaphoreType.DMA,
            pltpu.SemaphoreType.DMA,
        ],
    )


def kernel(x, edges, verts_idx, W0_1, b0_1, W1_1, b1_1, W0_2, b0_2,
           W1_2, b1_2, fc1_W, fc1_b, Wst, bst, Wse, bse, Wfu, bfu, Wae, bae):
    E = edges.shape[0]
    M = 2 * E
    cpt = -(-(-(-M // (NW * CH))) // 2) * 2
    mtot = NW * cpt * CH
    pad = mtot - M

    pad_src = jnp.arange(pad, dtype=jnp.int32) * 79 % N
    pad_dst = TRASH + (jnp.arange(pad, dtype=jnp.int32) % (NPAD - N))
    src = jnp.concatenate([edges[:, 0], edges[:, 1],
                           pad_src]).reshape(-1, 1, CH)
    dst = jnp.concatenate([edges[:, 1], edges[:, 0],
                           pad_dst]).reshape(-1, 1, CH)
    msg = jnp.concatenate([src, dst], axis=1)
    zeros = jnp.zeros((NPAD, D), jnp.float32)

    edge_agg = _make_edge_agg(cpt)

    v0, v1 = _lin2(x, W0_1, b0_1[None], W1_1, b1_1[None])
    agg1 = edge_agg(v1, msg, zeros)
    v0_2, v1_2 = _layer2(v0, agg1, W0_2, b0_2[None], W1_2, b1_2[None])
    agg2 = edge_agg(v1_2, msg, zeros)

    wcat = jnp.concatenate([Wst, Wse, Wfu, Wae], axis=1)
    wcat = jnp.pad(wcat, ((0, 0), (0, D - wcat.shape[1])))
    bcat = jnp.concatenate([bst, bse, bfu, bae])
    bcat = jnp.pad(bcat, (0, D - bcat.shape[0]))[None]

    outE = _pool_head(v0_2, agg2, verts_idx[None], fc1_W, fc1_b[None],
                      wcat, bcat)
    return (outE[:, 0:3], outE[:, 3:5], outE[:, 5:10], outE[:, 10:15])


# speedup vs baseline: 1.1314x; 1.1314x over previous
import functools

import jax
import jax.numpy as jnp
from jax import lax
from jax.experimental import pallas as pl
from jax.experimental.pallas import tpu as pltpu
from jax.experimental.pallas import tpu_sc as plsc

N = 10000
D = 128
B = 16
NC = 2
NS = 16
NW = NC * NS
CH = 128
NPAD = 10112
TRASH = N

_HI = lax.Precision.DEFAULT


_RB = 2000


def _lin2_body(x_ref, w0_ref, b0_ref, w1_ref, b1_ref, v0_ref, v1_ref):
    xb = x_ref[...]
    v0_ref[...] = jnp.dot(xb, w0_ref[...], preferred_element_type=jnp.float32,
                          precision=_HI) + b0_ref[...]
    v1_ref[...] = jnp.dot(xb, w1_ref[...], preferred_element_type=jnp.float32,
                          precision=_HI) + b1_ref[...]


_lin2 = pl.pallas_call(
    _lin2_body,
    grid=(N // _RB,),
    in_specs=[
        pl.BlockSpec((_RB, D), lambda i: (i, 0)),
        pl.BlockSpec((D, D), lambda i: (0, 0)),
        pl.BlockSpec((1, D), lambda i: (0, 0)),
        pl.BlockSpec((D, D), lambda i: (0, 0)),
        pl.BlockSpec((1, D), lambda i: (0, 0)),
    ],
    out_specs=[
        pl.BlockSpec((_RB, D), lambda i: (i, 0)),
        pl.BlockSpec((_RB, D), lambda i: (i, 0)),
    ],
    out_shape=[
        jax.ShapeDtypeStruct((N, D), jnp.float32),
        jax.ShapeDtypeStruct((N, D), jnp.float32),
    ],
)


def _layer2_body(v0_ref, agg_ref, w0_ref, b0_ref, w1_ref, b1_ref,
                 o0_ref, o1_ref):
    v = jnp.maximum(v0_ref[...] + agg_ref[0] + agg_ref[1], 0.0)
    o0_ref[...] = jnp.dot(v, w0_ref[...], preferred_element_type=jnp.float32,
                          precision=_HI) + b0_ref[...]
    o1_ref[...] = jnp.dot(v, w1_ref[...], preferred_element_type=jnp.float32,
                          precision=_HI) + b1_ref[...]


_layer2 = pl.pallas_call(
    _layer2_body,
    grid=(N // _RB,),
    in_specs=[
        pl.BlockSpec((_RB, D), lambda i: (i, 0)),
        pl.BlockSpec((NC, _RB, D), lambda i: (0, i, 0)),
        pl.BlockSpec((D, D), lambda i: (0, 0)),
        pl.BlockSpec((1, D), lambda i: (0, 0)),
        pl.BlockSpec((D, D), lambda i: (0, 0)),
        pl.BlockSpec((1, D), lambda i: (0, 0)),
    ],
    out_specs=[
        pl.BlockSpec((_RB, D), lambda i: (i, 0)),
        pl.BlockSpec((_RB, D), lambda i: (i, 0)),
    ],
    out_shape=[
        jax.ShapeDtypeStruct((N, D), jnp.float32),
        jax.ShapeDtypeStruct((N, D), jnp.float32),
    ],
)


def _pool_head_body(v0_ref, agg_ref, vi_ref, fc1w_ref, fc1b_ref,
                    wcat_ref, bcat_ref, out_ref):
    v2 = jnp.maximum(v0_ref[...] + agg_ref[0, :N] + agg_ref[1, :N], 0.0)
    vi = vi_ref[...]
    seg = lax.broadcasted_iota(jnp.int32, (B, N), 0)
    onehot = (seg == vi).astype(jnp.float32)
    counts = jnp.sum(onehot, axis=1)
    maxc = jnp.max(counts)
    pooled = jnp.dot(onehot, v2, preferred_element_type=jnp.float32,
                     precision=lax.Precision.HIGHEST) / maxc
    h = jnp.maximum(jnp.dot(pooled, fc1w_ref[...],
                            preferred_element_type=jnp.float32,
                            precision=_HI) + fc1b_ref[...], 0.0)
    out_ref[...] = jnp.dot(h, wcat_ref[...],
                           preferred_element_type=jnp.float32,
                           precision=_HI) + bcat_ref[...]


_pool_head = pl.pallas_call(
    _pool_head_body,
    out_shape=jax.ShapeDtypeStruct((B, D), jnp.float32),
)


def _edge_agg_body(cpt, v1_hbm, msg_hbm, zeros_hbm, out_hbm,
                   midx0, midx1, midx2, midx3, rows0, rows1, acc,
                   isem0, isem1, isem2, isem3, gsem0, gsem1, ssem0, ssem1):
    c = lax.axis_index("c")
    s = lax.axis_index("s")
    wid = c * NS + s
    zr = NPAD // NS
    pltpu.sync_copy(zeros_hbm.at[pl.ds(s * zr, zr)], acc.at[pl.ds(s * zr, zr)])
    plsc.subcore_barrier()

    midx = (midx0, midx1, midx2, midx3)
    rows = (rows0, rows1)
    isem = (isem0, isem1, isem2, isem3)
    gsem = (gsem0, gsem1)
    ssem = (ssem0, ssem1)
    base = wid * cpt

    pltpu.sync_copy(msg_hbm.at[base], midx0)
    pltpu.async_copy(msg_hbm.at[base + 1], midx1, isem1)
    pltpu.async_copy(v1_hbm.at[midx0.at[0]], rows0, gsem0)

    def step(k, i_val, first):
        q, nq, fq = k % 4, (k + 1) % 4, (k + 2) % 4
        r = k % 2
        nr = 1 - r
        pltpu.make_async_copy(msg_hbm.at[base], midx[nq], isem[nq]).wait()
        if not first:
            pltpu.make_async_copy(v1_hbm.at[pl.ds(0, CH)], rows[nr],
                                  ssem[nr]).wait()
        pltpu.async_copy(v1_hbm.at[midx[nq].at[0]], rows[nr], gsem[nr])
        pltpu.make_async_copy(v1_hbm.at[pl.ds(0, CH)], rows[r],
                              gsem[r]).wait()
        pltpu.async_copy(rows[r], acc.at[midx[q].at[1]], ssem[r], add=True)
        nxt = jnp.minimum(i_val + 2, cpt - 1)
        pltpu.async_copy(msg_hbm.at[base + nxt], midx[fq], isem[fq])

    step(0, 0, True)
    step(1, 1, False)
    step(2, 2, False)
    step(3, 3, False)

    def quad(j, carry):
        for k in range(4):
            step(k, 4 * j + k, False)
        return carry

    lax.fori_loop(1, cpt // 4, quad, 0)
    pltpu.make_async_copy(v1_hbm.at[pl.ds(0, CH)], rows0, gsem0).wait()
    pltpu.make_async_copy(v1_hbm.at[pl.ds(0, CH)], rows1, ssem1).wait()
    pltpu.make_async_copy(msg_hbm.at[base], midx1, isem1).wait()
    plsc.subcore_barrier()
    pltpu.sync_copy(acc.at[pl.ds(s * zr, zr)],
                    out_hbm.at[c].at[pl.ds(s * zr, zr)])


@functools.lru_cache(maxsize=None)
def _make_edge_agg(cpt):
    return pl.kernel(
        functools.partial(_edge_agg_body, cpt),
        out_type=jax.ShapeDtypeStruct((NC, NPAD, D), jnp.float32),
        mesh=plsc.VectorSubcoreMesh(core_axis_name="c", subcore_axis_name="s"),
        scratch_types=[
            pltpu.VMEM((2, CH), jnp.int32),
            pltpu.VMEM((2, CH), jnp.int32),
            pltpu.VMEM((2, CH), jnp.int32),
            pltpu.VMEM((2, CH), jnp.int32),
            pltpu.VMEM((CH, D), jnp.float32),
            pltpu.VMEM((CH, D), jnp.float32),
            pltpu.VMEM_SHARED((NPAD, D), jnp.float32),
            pltpu.SemaphoreType.DMA,
            pltpu.SemaphoreType.DMA,
            pltpu.SemaphoreType.DMA,
            pltpu.SemaphoreType.DMA,
            pltpu.SemaphoreType.DMA,
            pltpu.SemaphoreType.DMA,
            pltpu.SemaphoreType.DMA,
            pltpu.SemaphoreType.DMA,
        ],
    )


def kernel(x, edges, verts_idx, W0_1, b0_1, W1_1, b1_1, W0_2, b0_2,
           W1_2, b1_2, fc1_W, fc1_b, Wst, bst, Wse, bse, Wfu, bfu, Wae, bae):
    E = edges.shape[0]
    M = 2 * E
    cpt = -(-(-(-M // (NW * CH))) // 4) * 4
    mtot = NW * cpt * CH
    pad = mtot - M

    pad_src = jnp.arange(pad, dtype=jnp.int32) * 79 % N
    pad_dst = TRASH + (jnp.arange(pad, dtype=jnp.int32) % (NPAD - N))
    src = jnp.concatenate([edges[:, 0], edges[:, 1],
                           pad_src]).reshape(-1, 1, CH)
    dst = jnp.concatenate([edges[:, 1], edges[:, 0],
                           pad_dst]).reshape(-1, 1, CH)
    msg = jnp.concatenate([src, dst], axis=1)
    zeros = jnp.zeros((NPAD, D), jnp.float32)

    edge_agg = _make_edge_agg(cpt)

    v0, v1 = _lin2(x, W0_1, b0_1[None], W1_1, b1_1[None])
    agg1 = edge_agg(v1, msg, zeros)
    v0_2, v1_2 = _layer2(v0, agg1, W0_2, b0_2[None], W1_2, b1_2[None])
    agg2 = edge_agg(v1_2, msg, zeros)

    wcat = jnp.concatenate([Wst, Wse, Wfu, Wae], axis=1)
    wcat = jnp.pad(wcat, ((0, 0), (0, D - wcat.shape[1])))
    bcat = jnp.concatenate([bst, bse, bfu, bae])
    bcat = jnp.pad(bcat, (0, D - bcat.shape[0]))[None]

    outE = _pool_head(v0_2, agg2, verts_idx[None], fc1_W, fc1_b[None],
                      wcat, bcat)
    return (outE[:, 0:3], outE[:, 3:5], outE[:, 5:10], outE[:, 10:15])
